# baseline (speedup 1.0000x reference)
import functools

import jax
import jax.numpy as jnp
from jax import lax
from jax.experimental import pallas as pl
from jax.experimental.pallas import tpu as pltpu
from jax.experimental.pallas import tpu_sc as plsc

B = 16384
D = 64
K = 20
V = 1000000
NC = 2
NS = 16
NW = NC * NS
BPW = B // NW
GROUPS = 4
SUBS = 4

_mesh = plsc.VectorSubcoreMesh(core_axis_name="c", subcore_axis_name="s")


@functools.partial(
    pl.kernel,
    out_type=[
        jax.ShapeDtypeStruct((B,), jnp.float32),
        jax.ShapeDtypeStruct((B * K,), jnp.float32),
    ],
    mesh=_mesh,
    compiler_params=pltpu.CompilerParams(
        needs_layout_passes=False, use_tc_tiling_on_sc=True),
    scratch_types=[
        pltpu.VMEM((8, 128), jnp.int32),
        pltpu.VMEM((80, 128), jnp.int32),
        pltpu.VMEM((128, 128), jnp.float32),
        pltpu.VMEM((128, 128), jnp.float32),
        pltpu.VMEM((640, 128), jnp.float32),
        pltpu.VMEM((BPW,), jnp.float32),
        pltpu.VMEM((640,), jnp.float32),
        pltpu.SemaphoreType.DMA,
    ],
)
def _sc_scores(cp_ref, neg_ref, emb_ref,
               pos_out, neg_out,
               cp_idx, neg_idx, vbuf, ubuf, nbuf, psv, nsv, sem):
    wid = lax.axis_index("s") * NC + lax.axis_index("c")
    lanes = lax.iota(jnp.int32, 16)
    fifteen = jnp.full((16, 1), 15, jnp.int32)
    _gdn = lax.GatherDimensionNumbers(
        offset_dims=(), collapsed_slice_dims=(0,), start_index_map=(0,))

    def lanesum(t):
        return lax.gather(plsc.cumsum(t), fifteen, _gdn, (1,),
                          mode=lax.GatherScatterMode.PROMISE_IN_BOUNDS)

    pltpu.sync_copy(cp_ref.at[wid], cp_idx)
    pltpu.sync_copy(neg_ref.at[wid], neg_idx)

    def dot16(vrow, u_ref, urow):
        t = (vrow[0] * u_ref[urow, pl.ds(64, 16)]
             + vrow[1] * u_ref[urow, pl.ds(80, 16)]
             + vrow[2] * u_ref[urow, pl.ds(96, 16)]
             + vrow[3] * u_ref[urow, pl.ds(112, 16)])
        return lanesum(t)

    def vload(e):
        return [vbuf[e, pl.ds(16 * q, 16)] for q in range(4)]

    def do_group(j, carry):
        pltpu.async_copy(emb_ref.at[cp_idx.at[j]], vbuf, sem).wait()
        pltpu.async_copy(emb_ref.at[cp_idx.at[4 + j]], ubuf, sem).wait()

        def pos_body(g, c2):
            acc = jnp.zeros((16,), jnp.float32)
            for el in range(16):
                e = g * 16 + el
                acc = jnp.where(lanes == el, dot16(vload(e), ubuf, e), acc)
            psv[pl.ds(j * 128 + g * 16, 16)] = acc
            return c2
        lax.fori_loop(0, 8, pos_body, 0, unroll=False)

        def do_sub(su, c3):
            copies = []
            for q in range(5):
                copies.append(pltpu.async_copy(
                    emb_ref.at[neg_idx.at[j * 20 + su * 5 + q]],
                    nbuf.at[pl.ds(q * 128, 128)], sem))
            for cp in copies:
                cp.wait()

            def nbody(bi, c4):
                accs = [jnp.zeros((16,), jnp.float32) for _ in range(5)]
                for el in range(4):
                    e = su * 32 + bi * 4 + el
                    v = vload(e)
                    rbase = bi * 80 + el * 20
                    for k in range(20):
                        sc_i = el * 20 + k
                        accs[sc_i // 16] = jnp.where(
                            lanes == (sc_i % 16),
                            dot16(v, nbuf, rbase + k), accs[sc_i // 16])
                for w in range(5):
                    nsv[pl.ds(bi * 80 + w * 16, 16)] = accs[w]
                return c4
            lax.fori_loop(0, 8, nbody, 0, unroll=False)

            pltpu.sync_copy(
                nsv,
                neg_out.at[pl.ds((wid * BPW + j * 128 + su * 32) * K, 640)])
            return c3
        lax.fori_loop(0, SUBS, do_sub, 0, unroll=False)
        return carry

    lax.fori_loop(0, GROUPS, do_group, 0, unroll=False)
    pltpu.sync_copy(psv, pos_out.at[pl.ds(wid * BPW, BPW)])


_FUSE_ROWS = 8000


def _fuse_body(a_ref, b_ref, o_ref):
    o_ref[...] = jnp.concatenate([a_ref[...], b_ref[...]], axis=1)


_fuse_call = pl.pallas_call(
    _fuse_body,
    grid=(V // _FUSE_ROWS,),
    in_specs=[
        pl.BlockSpec((_FUSE_ROWS, D), lambda i: (i, 0)),
        pl.BlockSpec((_FUSE_ROWS, D), lambda i: (i, 0)),
    ],
    out_specs=pl.BlockSpec((_FUSE_ROWS, 2 * D), lambda i: (i, 0)),
    out_shape=jax.ShapeDtypeStruct((V, 2 * D), jnp.float32),
)


def _loss_body(pos_ref, neg_ref, out_ref):
    total = (jnp.sum(jax.nn.log_sigmoid(pos_ref[...]))
             + jnp.sum(jax.nn.log_sigmoid(-neg_ref[...])))
    out_ref[0, 0] = -total / B


_loss_call = pl.pallas_call(
    _loss_body,
    out_shape=jax.ShapeDtypeStruct((1, 1), jnp.float32),
    out_specs=pl.BlockSpec(memory_space=pltpu.SMEM),
)


def kernel(center_words, pos_words, neg_words, in_embed, out_embed):
    emb = _fuse_call(in_embed, out_embed)
    cen = center_words.astype(jnp.int32).reshape(NW, 4, 128)
    pos = pos_words.astype(jnp.int32).reshape(NW, 4, 128)
    cp = jnp.concatenate([cen, pos], axis=1)
    neg = neg_words.astype(jnp.int32).reshape(NW, 80, 128)
    pos_s, neg_s = _sc_scores(cp, neg, emb)
    out = _loss_call(pos_s.reshape(128, 128), neg_s.reshape(2560, 128))
    return out[0, 0]


# speedup vs baseline: 1.6060x; 1.6060x over previous
import functools

import jax
import jax.numpy as jnp
from jax import lax
from jax.experimental import pallas as pl
from jax.experimental.pallas import tpu as pltpu
from jax.experimental.pallas import tpu_sc as plsc

B = 16384
D = 64
K = 20
V = 1000000
NC = 2
NS = 16
NW = NC * NS
BPW = B // NW
GROUPS = 4
SUBS = 4

_mesh = plsc.VectorSubcoreMesh(core_axis_name="c", subcore_axis_name="s")


@functools.partial(
    pl.kernel,
    out_type=[
        jax.ShapeDtypeStruct((B,), jnp.float32),
        jax.ShapeDtypeStruct((B * K,), jnp.float32),
    ],
    mesh=_mesh,
    compiler_params=pltpu.CompilerParams(
        needs_layout_passes=False, use_tc_tiling_on_sc=True),
    scratch_types=[
        pltpu.VMEM((8, 128), jnp.int32),
        pltpu.VMEM((80, 128), jnp.int32),
        pltpu.VMEM((128, 128), jnp.float32),
        pltpu.VMEM((128, 128), jnp.float32),
        pltpu.VMEM((640, 128), jnp.float32),
        pltpu.VMEM((BPW,), jnp.float32),
        pltpu.VMEM((640,), jnp.float32),
        pltpu.SemaphoreType.DMA,
    ],
)
def _sc_scores(cp_ref, neg_ref, emb_ref,
               pos_out, neg_out,
               cp_idx, neg_idx, vbuf, ubuf, nbuf, psv, nsv, sem):
    wid = lax.axis_index("s") * NC + lax.axis_index("c")
    lanes = lax.iota(jnp.int32, 16)
    fifteen = jnp.full((16, 1), 15, jnp.int32)
    _gdn = lax.GatherDimensionNumbers(
        offset_dims=(), collapsed_slice_dims=(0,), start_index_map=(0,))

    def lanesum(t):
        return lax.gather(plsc.cumsum(t), fifteen, _gdn, (1,),
                          mode=lax.GatherScatterMode.PROMISE_IN_BOUNDS)

    pltpu.sync_copy(cp_ref.at[wid], cp_idx)
    pltpu.sync_copy(neg_ref.at[wid], neg_idx)

    def dot16(vrow, u_ref, urow):
        t = (vrow[0] * u_ref[urow, pl.ds(64, 16)]
             + vrow[1] * u_ref[urow, pl.ds(80, 16)]
             + vrow[2] * u_ref[urow, pl.ds(96, 16)]
             + vrow[3] * u_ref[urow, pl.ds(112, 16)])
        return lanesum(t)

    def vload(e):
        return [vbuf[e, pl.ds(16 * q, 16)] for q in range(4)]

    def do_group(j, carry):
        pltpu.async_copy(emb_ref.at[cp_idx.at[j]], vbuf, sem).wait()
        pltpu.async_copy(emb_ref.at[cp_idx.at[4 + j]], ubuf, sem).wait()

        def pos_body(g, c2):
            acc = jnp.zeros((16,), jnp.float32)
            for el in range(16):
                e = g * 16 + el
                acc = jnp.where(lanes == el, dot16(vload(e), ubuf, e), acc)
            psv[pl.ds(j * 128 + g * 16, 16)] = acc
            return c2
        lax.fori_loop(0, 8, pos_body, 0, unroll=False)

        def do_sub(su, c3):
            copies = []
            for q in range(5):
                copies.append(pltpu.async_copy(
                    emb_ref.at[neg_idx.at[j * 20 + su * 5 + q]],
                    nbuf.at[pl.ds(q * 128, 128)], sem))
            for cp in copies:
                cp.wait()

            def nbody(bi, c4):
                accs = [jnp.zeros((16,), jnp.float32) for _ in range(5)]
                for el in range(4):
                    e = su * 32 + bi * 4 + el
                    v = vload(e)
                    rbase = bi * 80 + el * 20
                    for k in range(20):
                        sc_i = el * 20 + k
                        accs[sc_i // 16] = jnp.where(
                            lanes == (sc_i % 16),
                            dot16(v, nbuf, rbase + k), accs[sc_i // 16])
                for w in range(5):
                    nsv[pl.ds(bi * 80 + w * 16, 16)] = accs[w]
                return c4
            lax.fori_loop(0, 8, nbody, 0, unroll=False)

            pltpu.sync_copy(
                nsv,
                neg_out.at[pl.ds((wid * BPW + j * 128 + su * 32) * K, 640)])
            return c3
        lax.fori_loop(0, SUBS, do_sub, 0, unroll=False)
        return carry

    lax.fori_loop(0, GROUPS, do_group, 0, unroll=False)
    pltpu.sync_copy(psv, pos_out.at[pl.ds(wid * BPW, BPW)])


_FUSE_ROWS = 2048


def _fuse_body(a_ref, b_ref, o_ref):
    r = lax.broadcasted_iota(jnp.int32, (D, D), 0)
    c = lax.broadcasted_iota(jnp.int32, (D, D), 1)
    ey = (r == c).astype(jnp.float32)
    dn = (((0,), (0,)), ((), ()))
    at = lax.dot_general(a_ref[...], ey, dn,
                         preferred_element_type=jnp.float32)
    bt = lax.dot_general(b_ref[...], ey, dn,
                         preferred_element_type=jnp.float32)
    o_ref[...] = jnp.concatenate([at, bt], axis=1)


_fuse_call = pl.pallas_call(
    _fuse_body,
    grid=(pl.cdiv(V, _FUSE_ROWS),),
    in_specs=[
        pl.BlockSpec((D, _FUSE_ROWS), lambda i: (0, i)),
        pl.BlockSpec((D, _FUSE_ROWS), lambda i: (0, i)),
    ],
    out_specs=pl.BlockSpec((_FUSE_ROWS, 2 * D), lambda i: (i, 0)),
    out_shape=jax.ShapeDtypeStruct((V, 2 * D), jnp.float32),
)


def _loss_body(pos_ref, neg_ref, out_ref):
    total = (jnp.sum(jax.nn.log_sigmoid(pos_ref[...]))
             + jnp.sum(jax.nn.log_sigmoid(-neg_ref[...])))
    out_ref[0, 0] = -total / B


_loss_call = pl.pallas_call(
    _loss_body,
    out_shape=jax.ShapeDtypeStruct((1, 1), jnp.float32),
    out_specs=pl.BlockSpec(memory_space=pltpu.SMEM),
)


def kernel(center_words, pos_words, neg_words, in_embed, out_embed):
    emb = _fuse_call(in_embed.T, out_embed.T)
    cen = center_words.astype(jnp.int32).reshape(NW, 4, 128)
    pos = pos_words.astype(jnp.int32).reshape(NW, 4, 128)
    cp = jnp.concatenate([cen, pos], axis=1)
    neg = neg_words.astype(jnp.int32).reshape(NW, 80, 128)
    pos_s, neg_s = _sc_scores(cp, neg, emb)
    out = _loss_call(pos_s.reshape(128, 128), neg_s.reshape(2560, 128))
    return out[0, 0]


# speedup vs baseline: 1.9296x; 1.2015x over previous
import functools

import jax
import jax.numpy as jnp
from jax import lax
from jax.experimental import pallas as pl
from jax.experimental.pallas import tpu as pltpu
from jax.experimental.pallas import tpu_sc as plsc

B = 16384
D = 64
K = 20
V = 1000000
NC = 2
NS = 16
NW = NC * NS
BPW = B // NW
GROUPS = 4
SUBS = 4

_mesh = plsc.VectorSubcoreMesh(core_axis_name="c", subcore_axis_name="s")


@functools.partial(
    pl.kernel,
    out_type=[
        jax.ShapeDtypeStruct((B,), jnp.float32),
        jax.ShapeDtypeStruct((B * K,), jnp.float32),
    ],
    mesh=_mesh,
    compiler_params=pltpu.CompilerParams(
        needs_layout_passes=False, use_tc_tiling_on_sc=True),
    scratch_types=[
        pltpu.VMEM((8, 128), jnp.int32),
        pltpu.VMEM((80, 128), jnp.int32),
        pltpu.VMEM((128, 128), jnp.float32),
        pltpu.VMEM((128, 128), jnp.float32),
        pltpu.VMEM((640, 128), jnp.float32),
        pltpu.VMEM((BPW,), jnp.float32),
        pltpu.VMEM((640,), jnp.float32),
        pltpu.SemaphoreType.DMA,
    ],
)
def _sc_scores(cp_ref, neg_ref, emb_ref,
               pos_out, neg_out,
               cp_idx, neg_idx, vbuf, ubuf, nbuf, psv, nsv, sem):
    wid = lax.axis_index("s") * NC + lax.axis_index("c")
    lanes = lax.iota(jnp.int32, 16)
    fifteen = jnp.full((16, 1), 15, jnp.int32)
    _gdn = lax.GatherDimensionNumbers(
        offset_dims=(), collapsed_slice_dims=(0,), start_index_map=(0,))

    def lanesum(t):
        return lax.gather(plsc.cumsum(t), fifteen, _gdn, (1,),
                          mode=lax.GatherScatterMode.PROMISE_IN_BOUNDS)

    pltpu.sync_copy(cp_ref.at[wid], cp_idx)
    pltpu.sync_copy(neg_ref.at[wid], neg_idx)

    def dot16(vrow, u_ref, urow):
        t = (vrow[0] * u_ref[urow, pl.ds(64, 16)]
             + vrow[1] * u_ref[urow, pl.ds(80, 16)]
             + vrow[2] * u_ref[urow, pl.ds(96, 16)]
             + vrow[3] * u_ref[urow, pl.ds(112, 16)])
        return lanesum(t)

    def vload(e):
        return [vbuf[e, pl.ds(16 * q, 16)] for q in range(4)]

    def do_group(j, carry):
        pltpu.async_copy(emb_ref.at[cp_idx.at[j]], vbuf, sem).wait()
        pltpu.async_copy(emb_ref.at[cp_idx.at[4 + j]], ubuf, sem).wait()

        def pos_body(g, c2):
            acc = jnp.zeros((16,), jnp.float32)
            for el in range(16):
                e = g * 16 + el
                acc = jnp.where(lanes == el, dot16(vload(e), ubuf, e), acc)
            psv[pl.ds(j * 128 + g * 16, 16)] = acc
            return c2
        lax.fori_loop(0, 8, pos_body, 0, unroll=False)

        def do_sub(su, c3):
            copies = []
            for q in range(5):
                copies.append(pltpu.async_copy(
                    emb_ref.at[neg_idx.at[j * 20 + su * 5 + q]],
                    nbuf.at[pl.ds(q * 128, 128)], sem))
            for cp in copies:
                cp.wait()

            def nbody(bi, c4):
                accs = [jnp.zeros((16,), jnp.float32) for _ in range(5)]
                for el in range(4):
                    e = su * 32 + bi * 4 + el
                    v = vload(e)
                    rbase = bi * 80 + el * 20
                    for k in range(20):
                        sc_i = el * 20 + k
                        accs[sc_i // 16] = jnp.where(
                            lanes == (sc_i % 16),
                            dot16(v, nbuf, rbase + k), accs[sc_i // 16])
                for w in range(5):
                    nsv[pl.ds(bi * 80 + w * 16, 16)] = accs[w]
                return c4
            lax.fori_loop(0, 8, nbody, 0, unroll=False)

            pltpu.sync_copy(
                nsv,
                neg_out.at[pl.ds((wid * BPW + j * 128 + su * 32) * K, 640)])
            return c3
        lax.fori_loop(0, SUBS, do_sub, 0, unroll=False)
        return carry

    lax.fori_loop(0, GROUPS, do_group, 0, unroll=False)
    pltpu.sync_copy(psv, pos_out.at[pl.ds(wid * BPW, BPW)])


_FUSE_ROWS = 4096


def _fuse_body(a_ref, b_ref, o_ref):
    at = a_ref[...].T
    bt = b_ref[...].T
    o_ref[...] = jnp.concatenate([at, bt], axis=1)


_fuse_call = pl.pallas_call(
    _fuse_body,
    grid=(pl.cdiv(V, _FUSE_ROWS),),
    in_specs=[
        pl.BlockSpec((D, _FUSE_ROWS), lambda i: (0, i)),
        pl.BlockSpec((D, _FUSE_ROWS), lambda i: (0, i)),
    ],
    out_specs=pl.BlockSpec((_FUSE_ROWS, 2 * D), lambda i: (i, 0)),
    out_shape=jax.ShapeDtypeStruct((V, 2 * D), jnp.float32),
)


def _loss_body(pos_ref, neg_ref, out_ref):
    total = (jnp.sum(jax.nn.log_sigmoid(pos_ref[...]))
             + jnp.sum(jax.nn.log_sigmoid(-neg_ref[...])))
    out_ref[0, 0] = -total / B


_loss_call = pl.pallas_call(
    _loss_body,
    out_shape=jax.ShapeDtypeStruct((1, 1), jnp.float32),
    out_specs=pl.BlockSpec(memory_space=pltpu.SMEM),
)


def kernel(center_words, pos_words, neg_words, in_embed, out_embed):
    emb = _fuse_call(in_embed.T, out_embed.T)
    cen = center_words.astype(jnp.int32).reshape(NW, 4, 128)
    pos = pos_words.astype(jnp.int32).reshape(NW, 4, 128)
    cp = jnp.concatenate([cen, pos], axis=1)
    neg = neg_words.astype(jnp.int32).reshape(NW, 80, 128)
    pos_s, neg_s = _sc_scores(cp, neg, emb)
    out = _loss_call(pos_s.reshape(128, 128), neg_s.reshape(2560, 128))
    return out[0, 0]


# speedup vs baseline: 2.1440x; 1.1111x over previous
import functools

import jax
import jax.numpy as jnp
from jax import lax
from jax.experimental import pallas as pl
from jax.experimental.pallas import tpu as pltpu
from jax.experimental.pallas import tpu_sc as plsc

B = 16384
D = 64
K = 20
V = 1000000
NC = 2
NS = 16
NW = NC * NS
BPW = B // NW
GROUPS = 4
SUBS = 4

_mesh = plsc.VectorSubcoreMesh(core_axis_name="c", subcore_axis_name="s")


@functools.partial(
    pl.kernel,
    out_type=[
        jax.ShapeDtypeStruct((B,), jnp.float32),
        jax.ShapeDtypeStruct((B * K,), jnp.float32),
    ],
    mesh=_mesh,
    compiler_params=pltpu.CompilerParams(
        needs_layout_passes=False, use_tc_tiling_on_sc=True),
    scratch_types=[
        pltpu.VMEM((8, 128), jnp.int32),
        pltpu.VMEM((80, 128), jnp.int32),
        pltpu.VMEM((128, 128), jnp.float32),
        pltpu.VMEM((128, 128), jnp.float32),
        pltpu.VMEM((640, 128), jnp.float32),
        pltpu.VMEM((BPW,), jnp.float32),
        pltpu.VMEM((640,), jnp.float32),
        pltpu.SemaphoreType.DMA,
    ],
)
def _sc_scores(cp_ref, neg_ref, emb_ref,
               pos_out, neg_out,
               cp_idx, neg_idx, vbuf, ubuf, nbuf, psv, nsv, sem):
    wid = lax.axis_index("s") * NC + lax.axis_index("c")
    lanes = lax.iota(jnp.int32, 16)
    fifteen = jnp.full((16, 1), 15, jnp.int32)
    _gdn = lax.GatherDimensionNumbers(
        offset_dims=(), collapsed_slice_dims=(0,), start_index_map=(0,))

    def lanesum(t):
        return lax.gather(plsc.cumsum(t), fifteen, _gdn, (1,),
                          mode=lax.GatherScatterMode.PROMISE_IN_BOUNDS)

    pltpu.sync_copy(cp_ref.at[wid], cp_idx)
    pltpu.sync_copy(neg_ref.at[wid], neg_idx)

    def dot16(vrow, u_ref, urow):
        t = (vrow[0] * u_ref[urow, pl.ds(64, 16)]
             + vrow[1] * u_ref[urow, pl.ds(80, 16)]
             + vrow[2] * u_ref[urow, pl.ds(96, 16)]
             + vrow[3] * u_ref[urow, pl.ds(112, 16)])
        return lanesum(t)

    def vload(e):
        return [vbuf[e, pl.ds(16 * q, 16)] for q in range(4)]

    def do_group(j, carry):
        pltpu.async_copy(emb_ref.at[cp_idx.at[j]], vbuf, sem).wait()
        pltpu.async_copy(emb_ref.at[cp_idx.at[4 + j]], ubuf, sem).wait()

        def pos_body(g, c2):
            acc = jnp.zeros((16,), jnp.float32)
            for el in range(16):
                e = g * 16 + el
                acc = jnp.where(lanes == el, dot16(vload(e), ubuf, e), acc)
            psv[pl.ds(j * 128 + g * 16, 16)] = acc
            return c2
        lax.fori_loop(0, 8, pos_body, 0, unroll=False)

        def do_sub(su, c3):
            copies = []
            for q in range(5):
                copies.append(pltpu.async_copy(
                    emb_ref.at[neg_idx.at[j * 20 + su * 5 + q]],
                    nbuf.at[pl.ds(q * 128, 128)], sem))
            for cp in copies:
                cp.wait()

            def nbody(bi, c4):
                accs = [jnp.zeros((16,), jnp.float32) for _ in range(5)]
                for el in range(4):
                    e = su * 32 + bi * 4 + el
                    v = vload(e)
                    rbase = bi * 80 + el * 20
                    for k in range(20):
                        sc_i = el * 20 + k
                        accs[sc_i // 16] = jnp.where(
                            lanes == (sc_i % 16),
                            dot16(v, nbuf, rbase + k), accs[sc_i // 16])
                for w in range(5):
                    nsv[pl.ds(bi * 80 + w * 16, 16)] = accs[w]
                return c4
            lax.fori_loop(0, 8, nbody, 0, unroll=False)

            pltpu.sync_copy(
                nsv,
                neg_out.at[pl.ds((wid * BPW + j * 128 + su * 32) * K, 640)])
            return c3
        lax.fori_loop(0, SUBS, do_sub, 0, unroll=False)
        return carry

    lax.fori_loop(0, GROUPS, do_group, 0, unroll=False)
    pltpu.sync_copy(psv, pos_out.at[pl.ds(wid * BPW, BPW)])


_FUSE_ROWS = 8192


def _fuse_body(a_ref, b_ref, o_ref):
    at = a_ref[...].T
    bt = b_ref[...].T
    o_ref[...] = jnp.concatenate([at, bt], axis=1)


_fuse_call = pl.pallas_call(
    _fuse_body,
    grid=(pl.cdiv(V, _FUSE_ROWS),),
    in_specs=[
        pl.BlockSpec((D, _FUSE_ROWS), lambda i: (0, i)),
        pl.BlockSpec((D, _FUSE_ROWS), lambda i: (0, i)),
    ],
    out_specs=pl.BlockSpec((_FUSE_ROWS, 2 * D), lambda i: (i, 0)),
    out_shape=jax.ShapeDtypeStruct((V, 2 * D), jnp.float32),
)


def _loss_body(pos_ref, neg_ref, out_ref):
    total = (jnp.sum(jax.nn.log_sigmoid(pos_ref[...]))
             + jnp.sum(jax.nn.log_sigmoid(-neg_ref[...])))
    out_ref[0, 0] = -total / B


_loss_call = pl.pallas_call(
    _loss_body,
    out_shape=jax.ShapeDtypeStruct((1, 1), jnp.float32),
    out_specs=pl.BlockSpec(memory_space=pltpu.SMEM),
)


def kernel(center_words, pos_words, neg_words, in_embed, out_embed):
    emb = _fuse_call(in_embed.T, out_embed.T)
    cen = center_words.astype(jnp.int32).reshape(NW, 4, 128)
    pos = pos_words.astype(jnp.int32).reshape(NW, 4, 128)
    cp = jnp.concatenate([cen, pos], axis=1)
    neg = neg_words.astype(jnp.int32).reshape(NW, 80, 128)
    pos_s, neg_s = _sc_scores(cp, neg, emb)
    out = _loss_call(pos_s.reshape(128, 128), neg_s.reshape(2560, 128))
    return out[0, 0]


# speedup vs baseline: 2.2614x; 1.0548x over previous
import functools

import jax
import jax.numpy as jnp
from jax import lax
from jax.experimental import pallas as pl
from jax.experimental.pallas import tpu as pltpu
from jax.experimental.pallas import tpu_sc as plsc

B = 16384
D = 64
K = 20
V = 1000000
NC = 2
NS = 16
NW = NC * NS
BPW = B // NW
GROUPS = 4
SUBS = 4

_mesh = plsc.VectorSubcoreMesh(core_axis_name="c", subcore_axis_name="s")


@functools.partial(
    pl.kernel,
    out_type=[
        jax.ShapeDtypeStruct((B,), jnp.float32),
        jax.ShapeDtypeStruct((B * K,), jnp.float32),
    ],
    mesh=_mesh,
    compiler_params=pltpu.CompilerParams(
        needs_layout_passes=False, use_tc_tiling_on_sc=True),
    scratch_types=[
        pltpu.VMEM((8, 128), jnp.int32),
        pltpu.VMEM((80, 128), jnp.int32),
        pltpu.VMEM((128, 128), jnp.float32),
        pltpu.VMEM((128, 128), jnp.float32),
        pltpu.VMEM((640, 128), jnp.float32),
        pltpu.VMEM((BPW,), jnp.float32),
        pltpu.VMEM((640,), jnp.float32),
        pltpu.SemaphoreType.DMA,
    ],
)
def _sc_scores(cp_ref, neg_ref, emb_ref,
               pos_out, neg_out,
               cp_idx, neg_idx, vbuf, ubuf, nbuf, psv, nsv, sem):
    wid = lax.axis_index("s") * NC + lax.axis_index("c")
    lanes = lax.iota(jnp.int32, 16)
    fifteen = jnp.full((16, 1), 15, jnp.int32)
    _gdn = lax.GatherDimensionNumbers(
        offset_dims=(), collapsed_slice_dims=(0,), start_index_map=(0,))

    def lanesum(t):
        return lax.gather(plsc.cumsum(t), fifteen, _gdn, (1,),
                          mode=lax.GatherScatterMode.PROMISE_IN_BOUNDS)

    pltpu.sync_copy(cp_ref.at[wid], cp_idx)
    pltpu.sync_copy(neg_ref.at[wid], neg_idx)

    def dot16(vrow, u_ref, urow):
        t = (vrow[0] * u_ref[urow, pl.ds(64, 16)]
             + vrow[1] * u_ref[urow, pl.ds(80, 16)]
             + vrow[2] * u_ref[urow, pl.ds(96, 16)]
             + vrow[3] * u_ref[urow, pl.ds(112, 16)])
        return lanesum(t)

    def vload(e):
        return [vbuf[e, pl.ds(16 * q, 16)] for q in range(4)]

    def do_group(j, carry):
        pltpu.async_copy(emb_ref.at[cp_idx.at[j]], vbuf, sem).wait()
        pltpu.async_copy(emb_ref.at[cp_idx.at[4 + j]], ubuf, sem).wait()

        def pos_body(g, c2):
            acc = jnp.zeros((16,), jnp.float32)
            for el in range(16):
                e = g * 16 + el
                acc = jnp.where(lanes == el, dot16(vload(e), ubuf, e), acc)
            psv[pl.ds(j * 128 + g * 16, 16)] = acc
            return c2
        lax.fori_loop(0, 8, pos_body, 0, unroll=False)

        def do_sub(su, c3):
            copies = []
            for q in range(5):
                copies.append(pltpu.async_copy(
                    emb_ref.at[neg_idx.at[j * 20 + su * 5 + q]],
                    nbuf.at[pl.ds(q * 128, 128)], sem))
            for cp in copies:
                cp.wait()

            def nbody(bi, c4):
                accs = [jnp.zeros((16,), jnp.float32) for _ in range(5)]
                for el in range(4):
                    e = su * 32 + bi * 4 + el
                    v = vload(e)
                    rbase = bi * 80 + el * 20
                    for k in range(20):
                        sc_i = el * 20 + k
                        accs[sc_i // 16] = jnp.where(
                            lanes == (sc_i % 16),
                            dot16(v, nbuf, rbase + k), accs[sc_i // 16])
                for w in range(5):
                    nsv[pl.ds(bi * 80 + w * 16, 16)] = accs[w]
                return c4
            lax.fori_loop(0, 8, nbody, 0, unroll=False)

            pltpu.sync_copy(
                nsv,
                neg_out.at[pl.ds((wid * BPW + j * 128 + su * 32) * K, 640)])
            return c3
        lax.fori_loop(0, SUBS, do_sub, 0, unroll=False)
        return carry

    lax.fori_loop(0, GROUPS, do_group, 0, unroll=False)
    pltpu.sync_copy(psv, pos_out.at[pl.ds(wid * BPW, BPW)])


_FUSE_ROWS = 16384


def _fuse_body(a_ref, b_ref, o_ref):
    at = a_ref[...].T
    bt = b_ref[...].T
    o_ref[...] = jnp.concatenate([at, bt], axis=1)


_fuse_call = pl.pallas_call(
    _fuse_body,
    grid=(pl.cdiv(V, _FUSE_ROWS),),
    in_specs=[
        pl.BlockSpec((D, _FUSE_ROWS), lambda i: (0, i)),
        pl.BlockSpec((D, _FUSE_ROWS), lambda i: (0, i)),
    ],
    out_specs=pl.BlockSpec((_FUSE_ROWS, 2 * D), lambda i: (i, 0)),
    out_shape=jax.ShapeDtypeStruct((V, 2 * D), jnp.float32),
)


def _loss_body(pos_ref, neg_ref, out_ref):
    total = (jnp.sum(jax.nn.log_sigmoid(pos_ref[...]))
             + jnp.sum(jax.nn.log_sigmoid(-neg_ref[...])))
    out_ref[0, 0] = -total / B


_loss_call = pl.pallas_call(
    _loss_body,
    out_shape=jax.ShapeDtypeStruct((1, 1), jnp.float32),
    out_specs=pl.BlockSpec(memory_space=pltpu.SMEM),
)


def kernel(center_words, pos_words, neg_words, in_embed, out_embed):
    emb = _fuse_call(in_embed.T, out_embed.T)
    cen = center_words.astype(jnp.int32).reshape(NW, 4, 128)
    pos = pos_words.astype(jnp.int32).reshape(NW, 4, 128)
    cp = jnp.concatenate([cen, pos], axis=1)
    neg = neg_words.astype(jnp.int32).reshape(NW, 80, 128)
    pos_s, neg_s = _sc_scores(cp, neg, emb)
    out = _loss_call(pos_s.reshape(128, 128), neg_s.reshape(2560, 128))
    return out[0, 0]


# speedup vs baseline: 2.5659x; 1.1346x over previous
import functools

import jax
import jax.numpy as jnp
from jax import lax
from jax.experimental import pallas as pl
from jax.experimental.pallas import tpu as pltpu
from jax.experimental.pallas import tpu_sc as plsc

B = 16384
D = 64
K = 20
V = 1000000
NC = 2
NS = 16
NW = NC * NS
BPW = B // NW
GROUPS = 4
SUBS = 4

_mesh = plsc.VectorSubcoreMesh(core_axis_name="c", subcore_axis_name="s")


@functools.partial(
    pl.kernel,
    out_type=[
        jax.ShapeDtypeStruct((B,), jnp.float32),
        jax.ShapeDtypeStruct((B * K,), jnp.float32),
    ],
    mesh=_mesh,
    compiler_params=pltpu.CompilerParams(
        needs_layout_passes=False, use_tc_tiling_on_sc=True),
    scratch_types=[
        pltpu.VMEM((8, 128), jnp.int32),
        pltpu.VMEM((80, 128), jnp.int32),
        pltpu.VMEM((128, 128), jnp.float32),
        pltpu.VMEM((128, 128), jnp.float32),
        pltpu.VMEM((640, 128), jnp.float32),
        pltpu.VMEM((BPW,), jnp.float32),
        pltpu.VMEM((640,), jnp.float32),
        pltpu.SemaphoreType.DMA,
    ],
)
def _sc_scores(cp_ref, neg_ref, emb_ref,
               pos_out, neg_out,
               cp_idx, neg_idx, vbuf, ubuf, nbuf, psv, nsv, sem):
    wid = lax.axis_index("s") * NC + lax.axis_index("c")
    lanes = lax.iota(jnp.int32, 16)
    fifteen = jnp.full((16, 1), 15, jnp.int32)
    _gdn = lax.GatherDimensionNumbers(
        offset_dims=(), collapsed_slice_dims=(0,), start_index_map=(0,))

    def lanesum(t):
        return lax.gather(plsc.cumsum(t), fifteen, _gdn, (1,),
                          mode=lax.GatherScatterMode.PROMISE_IN_BOUNDS)

    pltpu.sync_copy(cp_ref.at[wid], cp_idx)
    pltpu.sync_copy(neg_ref.at[wid], neg_idx)

    def dot16(vrow, u_ref, urow):
        t = (vrow[0] * u_ref[urow, pl.ds(64, 16)]
             + vrow[1] * u_ref[urow, pl.ds(80, 16)]
             + vrow[2] * u_ref[urow, pl.ds(96, 16)]
             + vrow[3] * u_ref[urow, pl.ds(112, 16)])
        return lanesum(t)

    def vload(e):
        return [vbuf[e, pl.ds(16 * q, 16)] for q in range(4)]

    def do_group(j, carry):
        pltpu.async_copy(emb_ref.at[cp_idx.at[j]], vbuf, sem).wait()
        pltpu.async_copy(emb_ref.at[cp_idx.at[4 + j]], ubuf, sem).wait()

        def pos_body(g, c2):
            acc = jnp.zeros((16,), jnp.float32)
            for el in range(16):
                e = g * 16 + el
                acc = jnp.where(lanes == el, dot16(vload(e), ubuf, e), acc)
            psv[pl.ds(j * 128 + g * 16, 16)] = acc
            return c2
        lax.fori_loop(0, 8, pos_body, 0, unroll=False)

        def do_sub(su, c3):
            copies = []
            for q in range(5):
                copies.append(pltpu.async_copy(
                    emb_ref.at[neg_idx.at[j * 20 + su * 5 + q]],
                    nbuf.at[pl.ds(q * 128, 128)], sem))
            for cp in copies:
                cp.wait()

            def nbody(bi, c4):
                accs = [jnp.zeros((16,), jnp.float32) for _ in range(5)]
                for el in range(4):
                    e = su * 32 + bi * 4 + el
                    v = vload(e)
                    rbase = bi * 80 + el * 20
                    for k in range(20):
                        sc_i = el * 20 + k
                        accs[sc_i // 16] = jnp.where(
                            lanes == (sc_i % 16),
                            dot16(v, nbuf, rbase + k), accs[sc_i // 16])
                for w in range(5):
                    nsv[pl.ds(bi * 80 + w * 16, 16)] = accs[w]
                return c4
            lax.fori_loop(0, 8, nbody, 0, unroll=False)

            pltpu.sync_copy(
                nsv,
                neg_out.at[pl.ds((wid * BPW + j * 128 + su * 32) * K, 640)])
            return c3
        lax.fori_loop(0, SUBS, do_sub, 0, unroll=False)
        return carry

    lax.fori_loop(0, GROUPS, do_group, 0, unroll=False)
    pltpu.sync_copy(psv, pos_out.at[pl.ds(wid * BPW, BPW)])


_FUSE_ROWS = 16384


def _fuse_body(a_ref, b_ref, o_ref):
    r = lax.broadcasted_iota(jnp.int32, (D, D), 0)
    c = lax.broadcasted_iota(jnp.int32, (D, D), 1)
    ey = (r == c).astype(jnp.bfloat16)
    dn = (((0,), (0,)), ((), ()))
    at = lax.dot_general(a_ref[...].astype(jnp.bfloat16), ey, dn,
                         preferred_element_type=jnp.float32)
    bt = lax.dot_general(b_ref[...].astype(jnp.bfloat16), ey, dn,
                         preferred_element_type=jnp.float32)
    o_ref[...] = jnp.concatenate([at, bt], axis=1)


_fuse_call = pl.pallas_call(
    _fuse_body,
    grid=(pl.cdiv(V, _FUSE_ROWS),),
    in_specs=[
        pl.BlockSpec((D, _FUSE_ROWS), lambda i: (0, i)),
        pl.BlockSpec((D, _FUSE_ROWS), lambda i: (0, i)),
    ],
    out_specs=pl.BlockSpec((_FUSE_ROWS, 2 * D), lambda i: (i, 0)),
    out_shape=jax.ShapeDtypeStruct((V, 2 * D), jnp.float32),
)


def _loss_body(pos_ref, neg_ref, out_ref):
    total = (jnp.sum(jax.nn.log_sigmoid(pos_ref[...]))
             + jnp.sum(jax.nn.log_sigmoid(-neg_ref[...])))
    out_ref[0, 0] = -total / B


_loss_call = pl.pallas_call(
    _loss_body,
    out_shape=jax.ShapeDtypeStruct((1, 1), jnp.float32),
    out_specs=pl.BlockSpec(memory_space=pltpu.SMEM),
)


def kernel(center_words, pos_words, neg_words, in_embed, out_embed):
    emb = _fuse_call(in_embed.T, out_embed.T)
    cen = center_words.astype(jnp.int32).reshape(NW, 4, 128)
    pos = pos_words.astype(jnp.int32).reshape(NW, 4, 128)
    cp = jnp.concatenate([cen, pos], axis=1)
    neg = neg_words.astype(jnp.int32).reshape(NW, 80, 128)
    pos_s, neg_s = _sc_scores(cp, neg, emb)
    out = _loss_call(pos_s.reshape(128, 128), neg_s.reshape(2560, 128))
    return out[0, 0]


# speedup vs baseline: 2.5800x; 1.0055x over previous
import functools

import jax
import jax.numpy as jnp
from jax import lax
from jax.experimental import pallas as pl
from jax.experimental.pallas import tpu as pltpu
from jax.experimental.pallas import tpu_sc as plsc

B = 16384
D = 64
K = 20
V = 1000000
NC = 2
NS = 16
NW = NC * NS
BPW = B // NW
GROUPS = 4
SUBS = 4

_mesh = plsc.VectorSubcoreMesh(core_axis_name="c", subcore_axis_name="s")


@functools.partial(
    pl.kernel,
    out_type=[
        jax.ShapeDtypeStruct((B,), jnp.float32),
        jax.ShapeDtypeStruct((B * K,), jnp.float32),
    ],
    mesh=_mesh,
    compiler_params=pltpu.CompilerParams(
        needs_layout_passes=False, use_tc_tiling_on_sc=True),
    scratch_types=[
        pltpu.VMEM((8, 128), jnp.int32),
        pltpu.VMEM((40, 128), jnp.int32),
        pltpu.VMEM((128, 128), jnp.float32),
        pltpu.VMEM((128, 128), jnp.float32),
        pltpu.VMEM((640, 128), jnp.float32),
        pltpu.VMEM((BPW,), jnp.float32),
        pltpu.VMEM((SUBS, 640), jnp.float32),
        pltpu.SemaphoreType.DMA,
        pltpu.SemaphoreType.DMA,
    ],
)
def _sc_scores(cp_ref, neg_ref, emb_ref,
               pos_out, neg_out,
               cp_idx, neg_idx, vbuf, ubuf, nbuf, psv, nsv, sem, osem):
    wid = lax.axis_index("s") * NC + lax.axis_index("c")
    lanes = lax.iota(jnp.int32, 16)
    fifteen = jnp.full((16, 1), 15, jnp.int32)
    _gdn = lax.GatherDimensionNumbers(
        offset_dims=(), collapsed_slice_dims=(0,), start_index_map=(0,))

    def lanesum(t):
        return lax.gather(plsc.cumsum(t), fifteen, _gdn, (1,),
                          mode=lax.GatherScatterMode.PROMISE_IN_BOUNDS)

    pltpu.sync_copy(cp_ref.at[wid], cp_idx)

    def dot16(vrow, u_ref, urow):
        t = (vrow[0] * u_ref[urow, pl.ds(64, 16)]
             + vrow[1] * u_ref[urow, pl.ds(80, 16)]
             + vrow[2] * u_ref[urow, pl.ds(96, 16)]
             + vrow[3] * u_ref[urow, pl.ds(112, 16)])
        return lanesum(t)

    def vload(e):
        return [vbuf[e, pl.ds(16 * q, 16)] for q in range(4)]

    def do_supergroup(j2, carry):
        pltpu.sync_copy(neg_ref.at[wid, pl.ds(j2 * 40, 40)], neg_idx)
        for j01 in range(2):
            _do_group(j2 * 2 + j01, j01)
        return carry

    def _do_group(j, j01):
        cpv = pltpu.async_copy(emb_ref.at[cp_idx.at[j]], vbuf, sem)
        cpu = pltpu.async_copy(emb_ref.at[cp_idx.at[4 + j]], ubuf, sem)
        cpv.wait()
        cpu.wait()

        def pos_body(g, c2):
            acc = jnp.zeros((16,), jnp.float32)
            for el in range(16):
                e = g * 16 + el
                acc = jnp.where(lanes == el, dot16(vload(e), ubuf, e), acc)
            psv[pl.ds(j * 128 + g * 16, 16)] = acc
            return c2
        lax.fori_loop(0, 8, pos_body, 0, unroll=False)

        def do_sub(su, c3):
            copies = []
            for q in range(5):
                copies.append(pltpu.async_copy(
                    emb_ref.at[neg_idx.at[j01 * 20 + su * 5 + q]],
                    nbuf.at[pl.ds(q * 128, 128)], sem))
            for cp in copies:
                cp.wait()

            def nbody(bi, c4):
                accs = [jnp.zeros((16,), jnp.float32) for _ in range(5)]
                for el in range(4):
                    e = su * 32 + bi * 4 + el
                    v = vload(e)
                    rbase = bi * 80 + el * 20
                    for k in range(20):
                        sc_i = el * 20 + k
                        accs[sc_i // 16] = jnp.where(
                            lanes == (sc_i % 16),
                            dot16(v, nbuf, rbase + k), accs[sc_i // 16])
                for w in range(5):
                    nsv[su, pl.ds(bi * 80 + w * 16, 16)] = accs[w]
                return c4
            lax.fori_loop(0, 8, nbody, 0, unroll=False)

            pltpu.async_copy(
                nsv.at[su],
                neg_out.at[pl.ds((wid * BPW + j * 128 + su * 32) * K, 640)],
                osem)
            return c3
        lax.fori_loop(0, SUBS, do_sub, 0, unroll=False)
        for su in range(SUBS):
            pltpu.make_async_copy(
                nsv.at[su],
                neg_out.at[pl.ds((wid * BPW + j * 128 + su * 32) * K, 640)],
                osem).wait()

    lax.fori_loop(0, GROUPS // 2, do_supergroup, 0, unroll=False)
    pltpu.sync_copy(psv, pos_out.at[pl.ds(wid * BPW, BPW)])


_FUSE_ROWS = 16384


def _fuse_body(a_ref, b_ref, o_ref):
    r = lax.broadcasted_iota(jnp.int32, (D, D), 0)
    c = lax.broadcasted_iota(jnp.int32, (D, D), 1)
    ey = (r == c).astype(jnp.bfloat16)
    dn = (((0,), (0,)), ((), ()))
    at = lax.dot_general(a_ref[...].astype(jnp.bfloat16), ey, dn,
                         preferred_element_type=jnp.float32)
    bt = lax.dot_general(b_ref[...].astype(jnp.bfloat16), ey, dn,
                         preferred_element_type=jnp.float32)
    o_ref[...] = jnp.concatenate([at, bt], axis=1)


_fuse_call = pl.pallas_call(
    _fuse_body,
    grid=(pl.cdiv(V, _FUSE_ROWS),),
    in_specs=[
        pl.BlockSpec((D, _FUSE_ROWS), lambda i: (0, i)),
        pl.BlockSpec((D, _FUSE_ROWS), lambda i: (0, i)),
    ],
    out_specs=pl.BlockSpec((_FUSE_ROWS, 2 * D), lambda i: (i, 0)),
    out_shape=jax.ShapeDtypeStruct((V, 2 * D), jnp.float32),
)


def _loss_body(pos_ref, neg_ref, out_ref):
    total = (jnp.sum(jax.nn.log_sigmoid(pos_ref[...]))
             + jnp.sum(jax.nn.log_sigmoid(-neg_ref[...])))
    out_ref[0, 0] = -total / B


_loss_call = pl.pallas_call(
    _loss_body,
    out_shape=jax.ShapeDtypeStruct((1, 1), jnp.float32),
    out_specs=pl.BlockSpec(memory_space=pltpu.SMEM),
)


def kernel(center_words, pos_words, neg_words, in_embed, out_embed):
    emb = _fuse_call(in_embed.T, out_embed.T)
    cen = center_words.astype(jnp.int32).reshape(NW, 4, 128)
    pos = pos_words.astype(jnp.int32).reshape(NW, 4, 128)
    cp = jnp.concatenate([cen, pos], axis=1)
    neg = neg_words.astype(jnp.int32).reshape(NW, 80, 128)
    pos_s, neg_s = _sc_scores(cp, neg, emb)
    out = _loss_call(pos_s.reshape(128, 128), neg_s.reshape(2560, 128))
    return out[0, 0]
